# baseline (speedup 1.0000x reference)
import jax
import jax.numpy as jnp
from jax import lax
from jax.experimental import pallas as pl
from jax.experimental.pallas import tpu as pltpu
from jax.experimental.pallas import tpu_sc as plsc

_D = 64
_NC, _NS = 2, 16
_NW = _NC * _NS
_CPW = _D // _NW
_L = 16


def _embed_body(xt_hbm, tT_hbm, out_hbm, trow, ib0, ib1, ob0, ob1,
                isem, osem0, osem1):
    wid = lax.axis_index("s") * _NC + lax.axis_index("c")
    hist = xt_hbm.shape[0]
    btot = xt_hbm.shape[1]
    nv = btot // _L
    ibufs = (ib0, ib1)
    obufs = (ob0, ob1)
    osems = (osem0, osem1)

    def wait_idx(q):
        pltpu.make_async_copy(xt_hbm.at[0], ibufs[q], isem).wait()

    def wait_write(q):
        pltpu.make_async_copy(out_hbm.at[0, 0, pl.ds(0, 32), 0, pl.ds(0, 128)],
                              obufs[q], osems[q]).wait()

    for cc in range(_CPW):
        c = wid * _CPW + cc
        cb = c // 8
        ci = c % 8
        pltpu.sync_copy(tT_hbm.at[c], trow)
        pltpu.async_copy(xt_hbm.at[0], ibufs[0], isem)

        def gather_h(h, q):
            wait_idx(q)

            @pl.when(h + 1 < hist)
            def _():
                pltpu.async_copy(xt_hbm.at[h + 1], ibufs[1 - q], isem)

            @pl.when(h >= 2)
            def _():
                wait_write(q)

            def gather_m(m, carry):
                idx16 = ibufs[q][pl.ds(m * _L, _L)]
                vals = plsc.load_gather(trow, [idx16])
                obufs[q][m // 8, pl.ds((m % 8) * _L, _L)] = vals
                return carry

            lax.fori_loop(0, nv, gather_m, 0)
            pltpu.async_copy(
                obufs[q],
                out_hbm.at[h, cb, pl.ds(0, 32), ci, pl.ds(0, 128)],
                osems[q])

        def step(i, carry):
            for q in range(2):
                gather_h(2 * i + q, q)
            return carry

        lax.fori_loop(0, hist // 2, step, 0)
        wait_write(0)
        wait_write(1)


def kernel(x, embed_mat):
    b, h = x.shape
    xt = x.astype(jnp.int32).T
    tT = embed_mat.T
    mesh = plsc.VectorSubcoreMesh(core_axis_name="c", subcore_axis_name="s",
                                  num_cores=_NC, num_subcores=_NS)
    y5 = pl.kernel(
        _embed_body,
        out_type=jax.ShapeDtypeStruct((h, 8, b // 128, 8, 128), jnp.float32),
        mesh=mesh,
        scratch_types=[
            pltpu.VMEM((embed_mat.shape[0],), jnp.float32),
            pltpu.VMEM((b,), jnp.int32),
            pltpu.VMEM((b,), jnp.int32),
            pltpu.VMEM((b // 128, 128), jnp.float32),
            pltpu.VMEM((b // 128, 128), jnp.float32),
            pltpu.SemaphoreType.DMA,
            pltpu.SemaphoreType.DMA,
            pltpu.SemaphoreType.DMA,
        ],
        compiler_params=pltpu.CompilerParams(use_tc_tiling_on_sc=False,
                                             needs_layout_passes=False),
    )(xt, tT)
    return y5.transpose(2, 4, 0, 1, 3).reshape(b, h, _D)


# speedup vs baseline: 1.5424x; 1.5424x over previous
import jax
import jax.numpy as jnp
from jax import lax
from jax.experimental import pallas as pl
from jax.experimental.pallas import tpu as pltpu
from jax.experimental.pallas import tpu_sc as plsc

_D = 64
_NC, _NS = 2, 16
_NW = _NC * _NS
_CPW = _D // _NW
_L = 16


def _embed_body(xt_hbm, tT_hbm, out_hbm, trow, ib0, ib1, ob0, ob1,
                isem, osem0, osem1):
    wid = lax.axis_index("s") * _NC + lax.axis_index("c")
    hist = xt_hbm.shape[0]
    btot = xt_hbm.shape[1]
    nv = btot // _L
    ibufs = (ib0, ib1)
    obufs = (ob0, ob1)
    osems = (osem0, osem1)

    def wait_idx(q):
        pltpu.make_async_copy(xt_hbm.at[0], ibufs[q], isem).wait()

    def wait_write(q):
        pltpu.make_async_copy(out_hbm.at[0, 0, pl.ds(0, 32), 0, pl.ds(0, 128)],
                              obufs[q], osems[q]).wait()

    for cc in range(_CPW):
        c = wid * _CPW + cc
        cb = c // 8
        ci = c % 8
        pltpu.sync_copy(tT_hbm.at[c], trow)
        pltpu.async_copy(xt_hbm.at[0], ibufs[0], isem)

        def gather_h(h, q):
            wait_idx(q)

            @pl.when(h + 1 < hist)
            def _():
                pltpu.async_copy(xt_hbm.at[h + 1], ibufs[1 - q], isem)

            @pl.when(h >= 2)
            def _():
                wait_write(q)

            def gather_row(bb, carry):
                base = bb * 128
                for t in range(8):
                    idx16 = ibufs[q][pl.ds(base + t * _L, _L)]
                    vals = plsc.load_gather(trow, [idx16])
                    obufs[q][bb, pl.ds(t * _L, _L)] = vals
                return carry

            lax.fori_loop(0, nv // 8, gather_row, 0)
            pltpu.async_copy(
                obufs[q],
                out_hbm.at[h, cb, pl.ds(0, 32), ci, pl.ds(0, 128)],
                osems[q])

        def step(i, carry):
            for q in range(2):
                gather_h(2 * i + q, q)
            return carry

        lax.fori_loop(0, hist // 2, step, 0)
        wait_write(0)
        wait_write(1)


def kernel(x, embed_mat):
    b, h = x.shape
    xt = x.astype(jnp.int32).T
    tT = embed_mat.T
    mesh = plsc.VectorSubcoreMesh(core_axis_name="c", subcore_axis_name="s",
                                  num_cores=_NC, num_subcores=_NS)
    y5 = pl.kernel(
        _embed_body,
        out_type=jax.ShapeDtypeStruct((h, 8, b // 128, 8, 128), jnp.float32),
        mesh=mesh,
        scratch_types=[
            pltpu.VMEM((embed_mat.shape[0],), jnp.float32),
            pltpu.VMEM((b,), jnp.int32),
            pltpu.VMEM((b,), jnp.int32),
            pltpu.VMEM((b // 128, 128), jnp.float32),
            pltpu.VMEM((b // 128, 128), jnp.float32),
            pltpu.SemaphoreType.DMA,
            pltpu.SemaphoreType.DMA,
            pltpu.SemaphoreType.DMA,
        ],
        compiler_params=pltpu.CompilerParams(use_tc_tiling_on_sc=False,
                                             needs_layout_passes=False),
    )(xt, tT)
    return y5.transpose(2, 4, 0, 1, 3).reshape(b, h, _D)
